# baseline (speedup 1.0000x reference)
import jax
import jax.numpy as jnp
from jax.experimental import pallas as pl
from jax.experimental.pallas import tpu as pltpu

_N = 4096
_BLK = 512
_G = _N // _BLK
_ETA = 1000000.0


def _body(Q_ref, AT_ref, xT_ref, yT_ref, x_ref, y_ref, c_ref, b_ref,
          il_ref, iu_ref, l_ref, u_ref, o_ref, acc_ref):
    i = pl.program_id(0)

    @pl.when(i == 0)
    def _init():
        acc_ref[0] = 0.0

    qx = jnp.sum(Q_ref[...] * xT_ref[...], axis=1, keepdims=True)
    aty = jnp.sum(AT_ref[...] * yT_ref[...], axis=1, keepdims=True)

    sl = pl.ds(i * _BLK, _BLK)
    xb = x_ref[sl, :]
    cb = c_ref[sl, :]

    pg = cb - aty + qx
    rc = (jnp.maximum(pg, 0.0) * il_ref[sl, :]
          - jnp.maximum(-pg, 0.0) * iu_ref[sl, :])
    rcc = jnp.sum(jnp.where(rc > 0.0, l_ref[sl, :], u_ref[sl, :]) * rc)
    contrib = (jnp.sum(xb * qx) + jnp.sum(cb * xb)
               - jnp.sum(b_ref[sl, :] * y_ref[sl, :]) - rcc)
    acc_ref[0] = acc_ref[0] + contrib

    @pl.when(i == _G - 1)
    def _fin():
        o_ref[...] = jnp.full((1, 1), jnp.abs(acc_ref[0]) / _ETA,
                              dtype=jnp.float32)


def kernel(Q, A, AT, b, c, x, y, Iy, il, iu, l, u):
    del A, Iy
    c2 = c[:, None]
    b2 = b[:, None]
    xT = x.reshape(1, _N)
    yT = y.reshape(1, _N)
    vec = pl.BlockSpec((_N, 1), lambda i: (0, 0))
    row = pl.BlockSpec((1, _N), lambda i: (0, 0))
    out = pl.pallas_call(
        _body,
        grid=(_G,),
        in_specs=[
            pl.BlockSpec((_BLK, _N), lambda i: (i, 0)),
            pl.BlockSpec((_BLK, _N), lambda i: (i, 0)),
            row, row,
            vec, vec, vec, vec, vec, vec, vec, vec,
        ],
        out_specs=pl.BlockSpec((1, 1), lambda i: (0, 0)),
        out_shape=jax.ShapeDtypeStruct((1, 1), jnp.float32),
        scratch_shapes=[pltpu.SMEM((1,), jnp.float32)],
        compiler_params=pltpu.CompilerParams(
            dimension_semantics=("arbitrary",)),
    )(Q, AT, xT, yT, x, y, c2, b2, il, iu, l, u)
    return out
